# baseline (speedup 1.0000x reference)
import functools

import jax
import jax.numpy as jnp
from jax import lax
from jax.experimental import pallas as pl
from jax.experimental.pallas import tpu as pltpu
from jax.experimental.pallas import tpu_sc as plsc

D_MODEL = 64
DP = 128
NUM_CORES = 2
NUM_SUBCORES = 16
NW = NUM_CORES * NUM_SUBCORES
CHUNK = 128
NB = 5
PERIOD = 10


@functools.cache
def _make_kernel(hist: int, batch: int):
    assert batch == NW * CHUNK
    mesh = plsc.VectorSubcoreMesh(core_axis_name="c", subcore_axis_name="s")

    @functools.partial(
        pl.kernel,
        mesh=mesh,
        out_type=jax.ShapeDtypeStruct((hist, D_MODEL, batch), jnp.float32),
        scratch_types=[
            pltpu.VMEM((hist, CHUNK), jnp.int32),
            pltpu.VMEM((NB, CHUNK, DP), jnp.float32),
            pltpu.VMEM((2, D_MODEL, CHUNK), jnp.float32),
            pltpu.SemaphoreType.DMA((NB,)),
            pltpu.SemaphoreType.DMA((2,)),
        ],
        compiler_params=pltpu.CompilerParams(needs_layout_passes=False),
    )
    def emb_kernel(idx_hbm, table_hbm, out_hbm, idx_v, rows, tbuf, gsem, tsem):
        wid = lax.axis_index("s") * NUM_CORES + lax.axis_index("c")
        lane0 = wid * CHUNK
        pltpu.sync_copy(idx_hbm.at[:, pl.ds(lane0, CHUNK)], idx_v)

        iotas = [lax.iota(jnp.int32, 16) + tg * 16 for tg in range(8)]

        def gather(l, b):
            return pltpu.make_async_copy(
                table_hbm.at[idx_v.at[l]], rows.at[b], gsem.at[b])

        def twrite(l, t):
            return pltpu.make_async_copy(
                tbuf.at[t], out_hbm.at[l, :, pl.ds(lane0, CHUNK)], tsem.at[t])

        def transpose(b, t):
            src = rows.at[b]
            dst = tbuf.at[t]

            def body(d, _):
                dvec = jnp.full((16,), d, jnp.int32)
                for tg in range(8):
                    v = plsc.load_gather(src, [iotas[tg], dvec])
                    dst[d, pl.ds(tg * 16, 16)] = v
                return ()

            lax.fori_loop(0, D_MODEL, body, (), unroll=4)

        for b in range(NB):
            gather(b, b).start()

        def outer(j0, _):
            for k in range(PERIOD):
                l = j0 * PERIOD + k
                b = k % NB
                t = k % 2
                gather(l, b).wait()

                @pl.when(l >= 2)
                def _():
                    twrite(l - 2, t).wait()

                transpose(b, t)
                twrite(l, t).start()

                @pl.when(l + NB < hist)
                def _():
                    gather(l + NB, b).start()
            return ()

        lax.fori_loop(0, hist // PERIOD, outer, (), unroll=False)

        for t in range(2):
            twrite(hist - 2 + t, t).wait()

    return emb_kernel


def kernel(inputs, table):
    batch, hist = inputs.shape
    idx_t = inputs.astype(jnp.int32).T
    table_p = jnp.pad(table, ((0, 0), (0, DP - table.shape[1])))
    out_t = _make_kernel(hist, batch)(idx_t, table_p)
    return out_t.transpose(2, 0, 1)


# speedup vs baseline: 1.4820x; 1.4820x over previous
import functools

import jax
import jax.numpy as jnp
from jax import lax
from jax.experimental import pallas as pl
from jax.experimental.pallas import tpu as pltpu
from jax.experimental.pallas import tpu_sc as plsc

D_MODEL = 64
DP = 128
NUM_CORES = 2
NUM_SUBCORES = 16
NW = NUM_CORES * NUM_SUBCORES
CHUNK = 128
NB = 5
PERIOD = 10


@functools.cache
def _make_kernel(hist: int, batch: int):
    assert batch == NW * CHUNK
    mesh = plsc.VectorSubcoreMesh(core_axis_name="c", subcore_axis_name="s")

    @functools.partial(
        pl.kernel,
        mesh=mesh,
        out_type=jax.ShapeDtypeStruct((hist, D_MODEL, batch), jnp.float32),
        scratch_types=[
            pltpu.VMEM((hist, CHUNK), jnp.int32),
            pltpu.VMEM((NB, CHUNK, DP), jnp.float32),
            pltpu.VMEM((2, D_MODEL, CHUNK), jnp.float32),
            pltpu.SemaphoreType.DMA((NB,)),
            pltpu.SemaphoreType.DMA((2,)),
        ],
        compiler_params=pltpu.CompilerParams(needs_layout_passes=False),
    )
    def emb_kernel(idx_hbm, table_hbm, out_hbm, idx_v, rows, tbuf, gsem, tsem):
        wid = lax.axis_index("s") * NUM_CORES + lax.axis_index("c")
        lane0 = wid * CHUNK
        pltpu.sync_copy(idx_hbm.at[:, pl.ds(lane0, CHUNK)], idx_v)

        iotas = [lax.iota(jnp.int32, 16) + tg * 16 for tg in range(8)]

        def gather(l, b):
            return pltpu.make_async_copy(
                table_hbm.at[idx_v.at[l]], rows.at[b], gsem.at[b])

        def twrite(l, t):
            return pltpu.make_async_copy(
                tbuf.at[t], out_hbm.at[l, :, pl.ds(lane0, CHUNK)], tsem.at[t])

        def transpose(b, t):
            src = rows.at[b]
            dst = tbuf.at[t]

            @plsc.parallel_loop(0, D_MODEL, unroll=8)
            def _(d):
                dvec = jnp.full((16,), d, jnp.int32)
                for tg in range(8):
                    v = plsc.load_gather(src, [iotas[tg], dvec])
                    dst[d, pl.ds(tg * 16, 16)] = v

        for b in range(NB):
            gather(b, b).start()

        def outer(j0, _):
            for k in range(PERIOD):
                l = j0 * PERIOD + k
                b = k % NB
                t = k % 2
                gather(l, b).wait()

                @pl.when(l >= 2)
                def _():
                    twrite(l - 2, t).wait()

                transpose(b, t)
                twrite(l, t).start()

                @pl.when(l + NB < hist)
                def _():
                    gather(l + NB, b).start()
            return ()

        lax.fori_loop(0, hist // PERIOD, outer, (), unroll=False)

        for t in range(2):
            twrite(hist - 2 + t, t).wait()

    return emb_kernel


def kernel(inputs, table):
    batch, hist = inputs.shape
    idx_t = inputs.astype(jnp.int32).T
    table_p = jnp.pad(table, ((0, 0), (0, DP - table.shape[1])))
    out_t = _make_kernel(hist, batch)(idx_t, table_p)
    return out_t.transpose(2, 0, 1)


# speedup vs baseline: 1.7491x; 1.1802x over previous
import functools

import jax
import jax.numpy as jnp
from jax import lax
from jax.experimental import pallas as pl
from jax.experimental.pallas import tpu as pltpu
from jax.experimental.pallas import tpu_sc as plsc

D_MODEL = 64
DP = 128
NUM_CORES = 2
NUM_SUBCORES = 16
NW = NUM_CORES * NUM_SUBCORES
CHUNK = 128
NB = 5


@functools.cache
def _make_kernel(total: int):
    per_w = total // NW
    n_chunks = per_w // CHUNK
    mesh = plsc.VectorSubcoreMesh(core_axis_name="c", subcore_axis_name="s")

    @functools.partial(
        pl.kernel,
        mesh=mesh,
        out_type=jax.ShapeDtypeStruct((total, DP), jnp.float32),
        scratch_types=[
            pltpu.VMEM((per_w,), jnp.int32),
            pltpu.VMEM((NB, CHUNK, DP), jnp.float32),
            pltpu.SemaphoreType.DMA((NB,)),
            pltpu.SemaphoreType.DMA((NB,)),
        ],
    )
    def emb_kernel(idx_hbm, table_hbm, out_hbm, idx_v, rows, gsem, wsem):
        wid = lax.axis_index("s") * NUM_CORES + lax.axis_index("c")
        base = wid * per_w
        pltpu.sync_copy(idx_hbm.at[pl.ds(base, per_w)], idx_v)

        def gather(j, b):
            return pltpu.make_async_copy(
                table_hbm.at[idx_v.at[pl.ds(j * CHUNK, CHUNK)]],
                rows.at[b], gsem.at[b])

        def write(j, b):
            return pltpu.make_async_copy(
                rows.at[b],
                out_hbm.at[pl.ds(base + j * CHUNK, CHUNK)],
                wsem.at[b])

        LG = 3
        WD = NB - LG

        for j in range(LG):
            gather(j, j).start()

        def outer(j0, _):
            for k in range(NB):
                j = j0 * NB + k
                gather(j, k).wait()
                write(j, k).start()

                @pl.when(j >= WD)
                def _():
                    write(j - WD, (k - WD) % NB).wait()

                @pl.when(j + LG < n_chunks)
                def _():
                    gather(j + LG, (k + LG) % NB).start()
            return ()

        lax.fori_loop(0, n_chunks // NB, outer, (), unroll=False)

        for j in range(n_chunks - WD, n_chunks):
            write(j, j % NB).wait()

    return emb_kernel


def kernel(inputs, table):
    batch, hist = inputs.shape
    total = batch * hist
    assert total % (NW * CHUNK) == 0
    idx = inputs.astype(jnp.int32).reshape(total)
    table_p = jnp.pad(table, ((0, 0), (0, DP - table.shape[1])))
    out = _make_kernel(total)(idx, table_p)
    return out[:, :D_MODEL].reshape(batch, hist, table.shape[1])


# speedup vs baseline: 1.7504x; 1.0007x over previous
import functools

import jax
import jax.numpy as jnp
from jax import lax
from jax.experimental import pallas as pl
from jax.experimental.pallas import tpu as pltpu
from jax.experimental.pallas import tpu_sc as plsc

D_MODEL = 64
DP = 128
NUM_CORES = 2
NUM_SUBCORES = 16
NW = NUM_CORES * NUM_SUBCORES
CHUNK = 160
NB = 5


@functools.cache
def _make_kernel(total: int):
    per_w = total // NW
    n_chunks = per_w // CHUNK
    mesh = plsc.VectorSubcoreMesh(core_axis_name="c", subcore_axis_name="s")

    @functools.partial(
        pl.kernel,
        mesh=mesh,
        out_type=jax.ShapeDtypeStruct((total, DP), jnp.float32),
        scratch_types=[
            pltpu.VMEM((per_w,), jnp.int32),
            pltpu.VMEM((NB, CHUNK, DP), jnp.float32),
            pltpu.SemaphoreType.DMA((NB,)),
            pltpu.SemaphoreType.DMA((NB,)),
        ],
    )
    def emb_kernel(idx_hbm, table_hbm, out_hbm, idx_v, rows, gsem, wsem):
        wid = lax.axis_index("s") * NUM_CORES + lax.axis_index("c")
        base = wid * per_w
        pltpu.sync_copy(idx_hbm.at[pl.ds(base, per_w)], idx_v)

        def gather(j, b):
            return pltpu.make_async_copy(
                table_hbm.at[idx_v.at[pl.ds(j * CHUNK, CHUNK)]],
                rows.at[b], gsem.at[b])

        def write(j, b):
            return pltpu.make_async_copy(
                rows.at[b],
                out_hbm.at[pl.ds(base + j * CHUNK, CHUNK)],
                wsem.at[b])

        LG = 3
        WD = NB - LG

        for j in range(LG):
            gather(j, j).start()

        def outer(j0, _):
            for k in range(NB):
                j = j0 * NB + k
                gather(j, k).wait()
                write(j, k).start()

                @pl.when(j >= WD)
                def _():
                    write(j - WD, (k - WD) % NB).wait()

                @pl.when(j + LG < n_chunks)
                def _():
                    gather(j + LG, (k + LG) % NB).start()
            return ()

        lax.fori_loop(0, n_chunks // NB, outer, (), unroll=False)

        for j in range(n_chunks - WD, n_chunks):
            write(j, j % NB).wait()

    return emb_kernel


def kernel(inputs, table):
    batch, hist = inputs.shape
    total = batch * hist
    assert total % (NW * CHUNK) == 0
    idx = inputs.astype(jnp.int32).reshape(total)
    table_p = jnp.pad(table, ((0, 0), (0, DP - table.shape[1])))
    out = _make_kernel(total)(idx, table_p)
    return out[:, :D_MODEL].reshape(batch, hist, table.shape[1])
